# baseline (speedup 1.0000x reference)
import functools

import jax
import jax.numpy as jnp
from jax import lax
from jax.experimental import pallas as pl
from jax.experimental.pallas import tpu as pltpu
from jax.experimental.pallas import tpu_sc as plsc

N_NODES = 100000
N_EDGES = 1600000
EMBED = 32
HIDDEN = 32
CLASSES = 16
N_GRAPHS = 512
HALF = 16

N_PAD = 102400
E_PAD = 1605632

NC, NS = 2, 16
NW = NC * NS
LANES = 16

RW = 128
TOK_PER_TILE = N_PAD // NW
TOK_ROWS = TOK_PER_TILE // RW
EDG_PER_TILE_DEG = E_PAD // NW
DEG_CHUNK = 3136
EROWS = E_PAD // RW
EROWS_PER_TILE = EROWS // NS
SUP = 4
NSUP = EROWS_PER_TILE // SUP
NODES_PER_TILE = N_PAD // NS
WB_CHUNK = 320

_sc_mesh = plsc.VectorSubcoreMesh(core_axis_name="c", subcore_axis_name="s")
_sc_params = pltpu.CompilerParams(use_tc_tiling_on_sc=False,
                                  needs_layout_passes=False)


@functools.partial(
    pl.kernel,
    out_type=[jax.ShapeDtypeStruct((N_PAD, EMBED), jnp.float32),
              jax.ShapeDtypeStruct((N_PAD,), jnp.float32),
              jax.ShapeDtypeStruct((N_PAD,), jnp.float32),
              jax.ShapeDtypeStruct((NW, N_PAD), jnp.float32)],
    mesh=_sc_mesh,
    scratch_types=[
        pltpu.VMEM((TOK_ROWS, RW), jnp.int32),
        pltpu.VMEM((2 * RW, EMBED), jnp.float32),
        pltpu.VMEM((DEG_CHUNK, ), jnp.int32),
        pltpu.VMEM((N_PAD,), jnp.float32),
        pltpu.VMEM((NODES_PER_TILE,), jnp.float32),
        pltpu.SemaphoreType.DMA,
        pltpu.SemaphoreType.DMA,
    ],
    compiler_params=_sc_params,
)
def _kpre(x2, dst_flat, emb, h0, deg_a, deg_b, degp, xb, rows, dstb, degv,
          rowb, gsem, wsem):
    c = lax.axis_index("c")
    s = lax.axis_index("s")
    w = c * NS + s

    pltpu.sync_copy(x2.at[pl.ds(w * TOK_ROWS, TOK_ROWS)], xb)
    g0 = pltpu.async_copy(emb.at[xb.at[0]], rows.at[pl.ds(0, RW)], gsem)

    def zero_body(i, carry):
        degv[pl.ds(i * LANES, LANES)] = jnp.zeros((LANES,), jnp.float32)
        return carry

    lax.fori_loop(0, N_PAD // LANES, zero_body, 0)
    g0.wait()

    def body(k, carry):
        p = lax.rem(k, 2)

        @pl.when(k < TOK_ROWS - 1)
        def _():
            pltpu.async_copy(emb.at[xb.at[k + 1]],
                             rows.at[pl.ds((1 - p) * RW, RW)], gsem)

        pltpu.async_copy(rows.at[pl.ds(p * RW, RW)],
                         h0.at[pl.ds(w * TOK_PER_TILE + k * RW, RW)], wsem)

        @pl.when(k < TOK_ROWS - 1)
        def _():
            pltpu.make_async_copy(emb.at[xb.at[k + 1]],
                                  rows.at[pl.ds((1 - p) * RW, RW)], gsem).wait()

        pltpu.make_async_copy(rows.at[pl.ds(p * RW, RW)],
                              h0.at[pl.ds(w * TOK_PER_TILE + k * RW, RW)],
                              wsem).wait()
        return carry

    lax.fori_loop(0, TOK_ROWS, body, 0)

    ones = jnp.ones((LANES,), jnp.float32)

    def chunk_body(t, carry):
        pltpu.sync_copy(
            dst_flat.at[pl.ds(w * EDG_PER_TILE_DEG + t * DEG_CHUNK, DEG_CHUNK)],
            dstb)

        def vec_body(i, c2):
            plsc.addupdate_scatter(degv, [dstb[pl.ds(i * LANES, LANES)]], ones)
            return c2

        lax.fori_loop(0, DEG_CHUNK // LANES, vec_body, 0)
        return carry

    lax.fori_loop(0, EDG_PER_TILE_DEG // DEG_CHUNK, chunk_body, 0)

    pltpu.sync_copy(degv, degp.at[w])
    plsc.subcore_barrier()
    base = s * NODES_PER_TILE
    pltpu.sync_copy(degp.at[c * NS, pl.ds(base, NODES_PER_TILE)],
                    degv.at[pl.ds(0, NODES_PER_TILE)])
    for t in range(1, NS):
        pltpu.sync_copy(degp.at[c * NS + t, pl.ds(base, NODES_PER_TILE)], rowb)

        def add_body(i, carry):
            sl = pl.ds(i * LANES, LANES)
            degv[sl] = degv[sl] + rowb[sl]
            return carry

        lax.fori_loop(0, NODES_PER_TILE // LANES, add_body, 0)

    @pl.when(c == 0)
    def _():
        pltpu.sync_copy(degv.at[pl.ds(0, NODES_PER_TILE)],
                        deg_a.at[pl.ds(base, NODES_PER_TILE)])

    @pl.when(c == 1)
    def _():
        pltpu.sync_copy(degv.at[pl.ds(0, NODES_PER_TILE)],
                        deg_b.at[pl.ds(base, NODES_PER_TILE)])


def _edge_phase(g_ref, out_ref, acc, srcb, dstb, rows, zbuf, gsem, ssem,
                src2, dst2, s):
    def zb(i, carry):
        zbuf[i, :] = jnp.zeros((LANES,), jnp.float32)
        return carry

    lax.fori_loop(0, WB_CHUNK, zb, 0)
    for k in range(NODES_PER_TILE // WB_CHUNK):
        pltpu.sync_copy(
            zbuf, acc.at[pl.ds(s * NODES_PER_TILE + k * WB_CHUNK, WB_CHUNK)])
    plsc.subcore_barrier()

    base_r = s * EROWS_PER_TILE
    pltpu.sync_copy(src2.at[pl.ds(base_r, SUP)], srcb.at[pl.ds(0, SUP)])
    pltpu.sync_copy(dst2.at[pl.ds(base_r, SUP)], dstb.at[pl.ds(0, SUP)])

    def sup_body(t, carry):
        p = lax.rem(t, 2)
        q = 1 - p
        gds = [pltpu.async_copy(g_ref.at[srcb.at[p * SUP + j]],
                                rows.at[pl.ds((p * SUP + j) * RW, RW)], gsem)
               for j in range(SUP)]

        @pl.when(t >= 1)
        def _():
            for j in range(SUP):
                pltpu.make_async_copy(
                    rows.at[pl.ds((q * SUP + j) * RW, RW)],
                    acc.at[dstb.at[q * SUP + j]], ssem).wait()

        @pl.when(t < NSUP - 1)
        def _():
            r1 = base_r + (t + 1) * SUP
            pltpu.sync_copy(src2.at[pl.ds(r1, SUP)],
                            srcb.at[pl.ds(q * SUP, SUP)])
            pltpu.sync_copy(dst2.at[pl.ds(r1, SUP)],
                            dstb.at[pl.ds(q * SUP, SUP)])

        for d in gds:
            d.wait()
        for j in range(SUP):
            pltpu.async_copy(rows.at[pl.ds((p * SUP + j) * RW, RW)],
                             acc.at[dstb.at[p * SUP + j]], ssem, add=True)
        return carry

    lax.fori_loop(0, NSUP, sup_body, 0)
    qf = (NSUP - 1) % 2
    for j in range(SUP):
        pltpu.make_async_copy(rows.at[pl.ds((qf * SUP + j) * RW, RW)],
                              acc.at[dstb.at[qf * SUP + j]], ssem).wait()
    plsc.subcore_barrier()

    for k in range(NODES_PER_TILE // WB_CHUNK):
        sl = pl.ds(s * NODES_PER_TILE + k * WB_CHUNK, WB_CHUNK)
        pltpu.sync_copy(acc.at[sl], zbuf)
        pltpu.sync_copy(zbuf, out_ref.at[sl])


@functools.partial(
    pl.kernel,
    out_type=[jax.ShapeDtypeStruct((N_PAD, HALF), jnp.float32),
              jax.ShapeDtypeStruct((N_PAD, HALF), jnp.float32)],
    mesh=_sc_mesh,
    scratch_types=[
        pltpu.VMEM_SHARED((N_PAD, HALF), jnp.float32),
        pltpu.VMEM((2 * SUP, RW), jnp.int32),
        pltpu.VMEM((2 * SUP, RW), jnp.int32),
        pltpu.VMEM((2 * SUP * RW, HALF), jnp.float32),
        pltpu.VMEM((WB_CHUNK, HALF), jnp.float32),
        pltpu.SemaphoreType.DMA,
        pltpu.SemaphoreType.DMA,
    ],
    compiler_params=_sc_params,
)
def _kedge(src2, dst2, g0, g1, s0, s1, acc, srcb, dstb, rows, zbuf, gsem, ssem):
    c = lax.axis_index("c")
    s = lax.axis_index("s")

    @pl.when(c == 0)
    def _():
        _edge_phase(g0, s0, acc, srcb, dstb, rows, zbuf, gsem, ssem,
                    src2, dst2, s)

    @pl.when(c == 1)
    def _():
        _edge_phase(g1, s1, acc, srcb, dstb, rows, zbuf, gsem, ssem,
                    src2, dst2, s)


BLK1 = 5120
BLK3 = 4000


def _mm1_body(dega_ref, degb_ref, h0_ref, w1_ref, glo_ref, ghi_ref, dinv_ref):
    deg = dega_ref[...] + degb_ref[...] + 1.0
    dinv = lax.rsqrt(deg)
    g = jnp.dot(h0_ref[...], w1_ref[...],
                preferred_element_type=jnp.float32) * dinv
    glo_ref[...] = g[:, :HALF]
    ghi_ref[...] = g[:, HALF:]
    dinv_ref[...] = dinv


_mm1 = pl.pallas_call(
    _mm1_body,
    grid=(N_PAD // BLK1,),
    in_specs=[pl.BlockSpec((BLK1, 1), lambda i: (i, 0)),
              pl.BlockSpec((BLK1, 1), lambda i: (i, 0)),
              pl.BlockSpec((BLK1, EMBED), lambda i: (i, 0)),
              pl.BlockSpec((EMBED, HIDDEN), lambda i: (0, 0))],
    out_specs=[pl.BlockSpec((BLK1, HALF), lambda i: (i, 0)),
               pl.BlockSpec((BLK1, HALF), lambda i: (i, 0)),
               pl.BlockSpec((BLK1, 1), lambda i: (i, 0))],
    out_shape=[jax.ShapeDtypeStruct((N_PAD, HALF), jnp.float32),
               jax.ShapeDtypeStruct((N_PAD, HALF), jnp.float32),
               jax.ShapeDtypeStruct((N_PAD, 1), jnp.float32)],
)


def _mm2_body(s0_ref, s1_ref, glo_ref, ghi_ref, dinv_ref, b1_ref, w2_ref,
              olo_ref, ohi_ref):
    dinv = dinv_ref[...]
    b = b1_ref[...]
    hl = jnp.maximum(dinv * (s0_ref[...] + glo_ref[...]) + b[:, :HALF], 0.0)
    hh = jnp.maximum(dinv * (s1_ref[...] + ghi_ref[...]) + b[:, HALF:], 0.0)
    h = jnp.concatenate([hl, hh], axis=1)
    g = jnp.dot(h, w2_ref[...], preferred_element_type=jnp.float32) * dinv
    olo_ref[...] = g[:, :HALF]
    ohi_ref[...] = g[:, HALF:]


_mm2 = pl.pallas_call(
    _mm2_body,
    grid=(N_PAD // BLK1,),
    in_specs=[pl.BlockSpec((BLK1, HALF), lambda i: (i, 0)),
              pl.BlockSpec((BLK1, HALF), lambda i: (i, 0)),
              pl.BlockSpec((BLK1, HALF), lambda i: (i, 0)),
              pl.BlockSpec((BLK1, HALF), lambda i: (i, 0)),
              pl.BlockSpec((BLK1, 1), lambda i: (i, 0)),
              pl.BlockSpec((1, HIDDEN), lambda i: (0, 0)),
              pl.BlockSpec((HIDDEN, HIDDEN), lambda i: (0, 0))],
    out_specs=[pl.BlockSpec((BLK1, HALF), lambda i: (i, 0)),
               pl.BlockSpec((BLK1, HALF), lambda i: (i, 0))],
    out_shape=[jax.ShapeDtypeStruct((N_PAD, HALF), jnp.float32),
               jax.ShapeDtypeStruct((N_PAD, HALF), jnp.float32)],
)


def _mm3_body(s0_ref, s1_ref, glo_ref, ghi_ref, dinv_ref, b2_ref, batch_ref,
              wc_ref, bc_ref, out_ref, sums, cnts):
    i = pl.program_id(0)

    @pl.when(i == 0)
    def _():
        sums[...] = jnp.zeros_like(sums)
        cnts[...] = jnp.zeros_like(cnts)

    dinv = dinv_ref[...]
    b = b2_ref[...]
    hl = jnp.maximum(dinv * (s0_ref[...] + glo_ref[...]) + b[:, :HALF], 0.0)
    hh = jnp.maximum(dinv * (s1_ref[...] + ghi_ref[...]) + b[:, HALF:], 0.0)
    h = jnp.concatenate([hl, hh], axis=1)
    bb = batch_ref[...]
    gid = lax.broadcasted_iota(jnp.int32, (BLK3, N_GRAPHS), 1)
    a = (bb == gid).astype(jnp.float32)
    sums[...] += lax.dot_general(a, h, (((0,), (0,)), ((), ())),
                                 preferred_element_type=jnp.float32)
    cnts[...] += jnp.sum(a, axis=0)[:, None]

    @pl.when(i == N_NODES // BLK3 - 1)
    def _():
        pooled = sums[...] / jnp.maximum(cnts[...], 1.0)
        out_ref[...] = jnp.dot(pooled, wc_ref[...],
                               preferred_element_type=jnp.float32) + bc_ref[...]


_mm3 = pl.pallas_call(
    _mm3_body,
    grid=(N_NODES // BLK3,),
    in_specs=[pl.BlockSpec((BLK3, HALF), lambda i: (i, 0)),
              pl.BlockSpec((BLK3, HALF), lambda i: (i, 0)),
              pl.BlockSpec((BLK3, HALF), lambda i: (i, 0)),
              pl.BlockSpec((BLK3, HALF), lambda i: (i, 0)),
              pl.BlockSpec((BLK3, 1), lambda i: (i, 0)),
              pl.BlockSpec((1, HIDDEN), lambda i: (0, 0)),
              pl.BlockSpec((BLK3, 1), lambda i: (i, 0)),
              pl.BlockSpec((HIDDEN, CLASSES), lambda i: (0, 0)),
              pl.BlockSpec((1, CLASSES), lambda i: (0, 0))],
    out_specs=pl.BlockSpec((N_GRAPHS, CLASSES), lambda i: (0, 0)),
    out_shape=jax.ShapeDtypeStruct((N_GRAPHS, CLASSES), jnp.float32),
    scratch_shapes=[pltpu.VMEM((N_GRAPHS, HIDDEN), jnp.float32),
                    pltpu.VMEM((N_GRAPHS, 1), jnp.float32)],
    compiler_params=pltpu.CompilerParams(
        dimension_semantics=("arbitrary",)),
)


def kernel(x, edge_index, batch, emb, W1, b1, W2, b2, Wc, bc):
    x2 = jnp.pad(x, (0, N_PAD - N_NODES)).reshape(N_PAD // RW, RW)
    ei = jnp.pad(edge_index, ((0, 0), (0, E_PAD - N_EDGES)),
                 constant_values=N_NODES)
    src2 = ei[0].reshape(EROWS, RW)
    dst2 = ei[1].reshape(EROWS, RW)

    h0, deg_a, deg_b, _unused_degp = _kpre(x2, ei[1], emb)
    glo, ghi, dinv = _mm1(deg_a.reshape(N_PAD, 1), deg_b.reshape(N_PAD, 1),
                          h0, W1)
    s0, s1 = _kedge(src2, dst2, glo, ghi)
    g2lo, g2hi = _mm2(s0, s1, glo, ghi, dinv, b1.reshape(1, HIDDEN), W2)
    t0, t1 = _kedge(src2, dst2, g2lo, g2hi)
    out = _mm3(t0, t1, g2lo, g2hi, dinv, b2.reshape(1, HIDDEN),
               batch.reshape(N_NODES, 1), Wc, bc.reshape(1, CLASSES))
    return out


# speedup vs baseline: 1.3499x; 1.3499x over previous
import functools

import jax
import jax.numpy as jnp
from jax import lax
from jax.experimental import pallas as pl
from jax.experimental.pallas import tpu as pltpu
from jax.experimental.pallas import tpu_sc as plsc

N_NODES = 100000
N_EDGES = 1600000
EMBED = 32
HIDDEN = 32
CLASSES = 16
N_GRAPHS = 512
HALF = 16

N_PAD = 102400
E_PAD = 1605632
VOCAB_ROWS = 100000

NC, NS = 2, 16
NW = NC * NS
LANES = 16

RW = 128
TOK_PER_TILE = N_PAD // NW
TOK_ROWS = TOK_PER_TILE // RW
EDG_PER_TILE_DEG = E_PAD // NW
DEG_CHUNK = 3136
EROWS = E_PAD // RW
EROWS_PER_TILE = EROWS // NS
SUP = 4
NSUP = EROWS_PER_TILE // SUP
NODES_PER_TILE = N_PAD // NS
WB_CHUNK = 320

_sc_mesh = plsc.VectorSubcoreMesh(core_axis_name="c", subcore_axis_name="s")
_sc_params = pltpu.CompilerParams(use_tc_tiling_on_sc=False,
                                  needs_layout_passes=False)


@functools.partial(
    pl.kernel,
    out_type=[jax.ShapeDtypeStruct((N_PAD, HALF), jnp.float32),
              jax.ShapeDtypeStruct((N_PAD, HALF), jnp.float32),
              jax.ShapeDtypeStruct((N_PAD,), jnp.float32),
              jax.ShapeDtypeStruct((N_PAD,), jnp.float32),
              jax.ShapeDtypeStruct((NW, N_PAD), jnp.float32)],
    mesh=_sc_mesh,
    scratch_types=[
        pltpu.VMEM((TOK_ROWS, RW), jnp.int32),
        pltpu.VMEM((TOK_ROWS, RW), jnp.int32),
        pltpu.VMEM((TOK_ROWS, RW), jnp.int32),
        pltpu.VMEM((2 * RW, HALF), jnp.float32),
        pltpu.VMEM((2 * RW, HALF), jnp.float32),
        pltpu.VMEM((DEG_CHUNK, ), jnp.int32),
        pltpu.VMEM((N_PAD,), jnp.float32),
        pltpu.VMEM((NODES_PER_TILE,), jnp.float32),
        pltpu.SemaphoreType.DMA,
        pltpu.SemaphoreType.DMA,
    ],
    compiler_params=_sc_params,
)
def _kpre(x2, dst_flat, embv, h0lo, h0hi, deg_a, deg_b, degp, xb, xlo, xhi,
          rowsl, rowsh, dstb, degv, rowb, gsem, wsem):
    c = lax.axis_index("c")
    s = lax.axis_index("s")
    w = c * NS + s

    pltpu.sync_copy(x2.at[pl.ds(w * TOK_ROWS, TOK_ROWS)], xb)

    def idx_body(k, carry):
        def lane_body(j, c2):
            sl = pl.ds(j * LANES, LANES)
            v2 = xb[k, sl] * 2
            xlo[k, sl] = v2
            xhi[k, sl] = v2 + 1
            return c2

        lax.fori_loop(0, RW // LANES, lane_body, 0)
        return carry

    lax.fori_loop(0, TOK_ROWS, idx_body, 0)

    g0 = pltpu.async_copy(embv.at[xlo.at[0]], rowsl.at[pl.ds(0, RW)], gsem)
    g1 = pltpu.async_copy(embv.at[xhi.at[0]], rowsh.at[pl.ds(0, RW)], gsem)

    def zero_body(i, carry):
        degv[pl.ds(i * LANES, LANES)] = jnp.zeros((LANES,), jnp.float32)
        return carry

    lax.fori_loop(0, N_PAD // LANES, zero_body, 0)
    g0.wait()
    g1.wait()

    def body(k, carry):
        p = lax.rem(k, 2)
        hsl = pl.ds(w * TOK_PER_TILE + k * RW, RW)
        bsl = pl.ds(p * RW, RW)
        nsl = pl.ds((1 - p) * RW, RW)

        @pl.when(k < TOK_ROWS - 1)
        def _():
            pltpu.async_copy(embv.at[xlo.at[k + 1]], rowsl.at[nsl], gsem)
            pltpu.async_copy(embv.at[xhi.at[k + 1]], rowsh.at[nsl], gsem)

        pltpu.async_copy(rowsl.at[bsl], h0lo.at[hsl], wsem)
        pltpu.async_copy(rowsh.at[bsl], h0hi.at[hsl], wsem)

        @pl.when(k < TOK_ROWS - 1)
        def _():
            pltpu.make_async_copy(embv.at[xlo.at[k + 1]], rowsl.at[nsl],
                                  gsem).wait()
            pltpu.make_async_copy(embv.at[xhi.at[k + 1]], rowsh.at[nsl],
                                  gsem).wait()

        pltpu.make_async_copy(rowsl.at[bsl], h0lo.at[hsl], wsem).wait()
        pltpu.make_async_copy(rowsh.at[bsl], h0hi.at[hsl], wsem).wait()
        return carry

    lax.fori_loop(0, TOK_ROWS, body, 0)

    ones = jnp.ones((LANES,), jnp.float32)

    def chunk_body(t, carry):
        pltpu.sync_copy(
            dst_flat.at[pl.ds(w * EDG_PER_TILE_DEG + t * DEG_CHUNK, DEG_CHUNK)],
            dstb)

        def vec_body(i, c2):
            plsc.addupdate_scatter(degv, [dstb[pl.ds(i * LANES, LANES)]], ones)
            return c2

        lax.fori_loop(0, DEG_CHUNK // LANES, vec_body, 0)
        return carry

    lax.fori_loop(0, EDG_PER_TILE_DEG // DEG_CHUNK, chunk_body, 0)

    pltpu.sync_copy(degv, degp.at[w])
    plsc.subcore_barrier()
    base = s * NODES_PER_TILE
    pltpu.sync_copy(degp.at[c * NS, pl.ds(base, NODES_PER_TILE)],
                    degv.at[pl.ds(0, NODES_PER_TILE)])
    for t in range(1, NS):
        pltpu.sync_copy(degp.at[c * NS + t, pl.ds(base, NODES_PER_TILE)], rowb)

        def add_body(i, carry):
            sl = pl.ds(i * LANES, LANES)
            degv[sl] = degv[sl] + rowb[sl]
            return carry

        lax.fori_loop(0, NODES_PER_TILE // LANES, add_body, 0)

    @pl.when(c == 0)
    def _():
        pltpu.sync_copy(degv.at[pl.ds(0, NODES_PER_TILE)],
                        deg_a.at[pl.ds(base, NODES_PER_TILE)])

    @pl.when(c == 1)
    def _():
        pltpu.sync_copy(degv.at[pl.ds(0, NODES_PER_TILE)],
                        deg_b.at[pl.ds(base, NODES_PER_TILE)])


def _edge_phase(g_ref, out_ref, acc, srcb, dstb, rows, zbuf, gsem, ssem,
                src2, dst2, s):
    def zb(i, carry):
        zbuf[i, :] = jnp.zeros((LANES,), jnp.float32)
        return carry

    lax.fori_loop(0, WB_CHUNK, zb, 0)
    for k in range(NODES_PER_TILE // WB_CHUNK):
        pltpu.sync_copy(
            zbuf, acc.at[pl.ds(s * NODES_PER_TILE + k * WB_CHUNK, WB_CHUNK)])
    plsc.subcore_barrier()

    base_r = s * EROWS_PER_TILE
    pltpu.sync_copy(src2.at[pl.ds(base_r, SUP)], srcb.at[pl.ds(0, SUP)])
    pltpu.sync_copy(dst2.at[pl.ds(base_r, SUP)], dstb.at[pl.ds(0, SUP)])

    def sup_body(t, carry):
        p = lax.rem(t, 2)
        q = 1 - p
        gds = [pltpu.async_copy(g_ref.at[srcb.at[p * SUP + j]],
                                rows.at[pl.ds((p * SUP + j) * RW, RW)], gsem)
               for j in range(SUP)]

        @pl.when(t >= 1)
        def _():
            for j in range(SUP):
                pltpu.make_async_copy(
                    rows.at[pl.ds((q * SUP + j) * RW, RW)],
                    acc.at[dstb.at[q * SUP + j]], ssem).wait()

        @pl.when(t < NSUP - 1)
        def _():
            r1 = base_r + (t + 1) * SUP
            pltpu.sync_copy(src2.at[pl.ds(r1, SUP)],
                            srcb.at[pl.ds(q * SUP, SUP)])
            pltpu.sync_copy(dst2.at[pl.ds(r1, SUP)],
                            dstb.at[pl.ds(q * SUP, SUP)])

        for d in gds:
            d.wait()
        for j in range(SUP):
            pltpu.async_copy(rows.at[pl.ds((p * SUP + j) * RW, RW)],
                             acc.at[dstb.at[p * SUP + j]], ssem, add=True)
        return carry

    lax.fori_loop(0, NSUP, sup_body, 0)
    qf = (NSUP - 1) % 2
    for j in range(SUP):
        pltpu.make_async_copy(rows.at[pl.ds((qf * SUP + j) * RW, RW)],
                              acc.at[dstb.at[qf * SUP + j]], ssem).wait()
    plsc.subcore_barrier()

    for k in range(NODES_PER_TILE // WB_CHUNK):
        sl = pl.ds(s * NODES_PER_TILE + k * WB_CHUNK, WB_CHUNK)
        pltpu.sync_copy(acc.at[sl], zbuf)
        pltpu.sync_copy(zbuf, out_ref.at[sl])


@functools.partial(
    pl.kernel,
    out_type=[jax.ShapeDtypeStruct((N_PAD, HALF), jnp.float32),
              jax.ShapeDtypeStruct((N_PAD, HALF), jnp.float32)],
    mesh=_sc_mesh,
    scratch_types=[
        pltpu.VMEM_SHARED((N_PAD, HALF), jnp.float32),
        pltpu.VMEM((2 * SUP, RW), jnp.int32),
        pltpu.VMEM((2 * SUP, RW), jnp.int32),
        pltpu.VMEM((2 * SUP * RW, HALF), jnp.float32),
        pltpu.VMEM((WB_CHUNK, HALF), jnp.float32),
        pltpu.SemaphoreType.DMA,
        pltpu.SemaphoreType.DMA,
    ],
    compiler_params=_sc_params,
)
def _kedge(src2, dst2, g0, g1, s0, s1, acc, srcb, dstb, rows, zbuf, gsem, ssem):
    c = lax.axis_index("c")
    s = lax.axis_index("s")

    @pl.when(c == 0)
    def _():
        _edge_phase(g0, s0, acc, srcb, dstb, rows, zbuf, gsem, ssem,
                    src2, dst2, s)

    @pl.when(c == 1)
    def _():
        _edge_phase(g1, s1, acc, srcb, dstb, rows, zbuf, gsem, ssem,
                    src2, dst2, s)


FLAT = N_PAD // 8
BLKF = FLAT // 4
BLK3 = 4000


def _mm1_body(dega_ref, degb_ref, h0lo_ref, h0hi_ref, e8_ref, w1a_ref,
              w1b_ref, w1c_ref, w1d_ref, glo_ref, ghi_ref, dinv_ref):
    deg = dega_ref[...] + degb_ref[...] + 1.0
    dinv8 = lax.rsqrt(deg)
    dinvf = jnp.dot(dinv8, e8_ref[...], preferred_element_type=jnp.float32)
    hl = h0lo_ref[...]
    hh = h0hi_ref[...]
    glo_ref[...] = (jnp.dot(hl, w1a_ref[...], preferred_element_type=jnp.float32)
                    + jnp.dot(hh, w1b_ref[...],
                              preferred_element_type=jnp.float32)) * dinvf
    ghi_ref[...] = (jnp.dot(hl, w1c_ref[...], preferred_element_type=jnp.float32)
                    + jnp.dot(hh, w1d_ref[...],
                              preferred_element_type=jnp.float32)) * dinvf
    dinv_ref[...] = dinv8


_mm1 = pl.pallas_call(
    _mm1_body,
    grid=(FLAT // BLKF,),
    in_specs=[pl.BlockSpec((BLKF, 8), lambda i: (i, 0)),
              pl.BlockSpec((BLKF, 8), lambda i: (i, 0)),
              pl.BlockSpec((BLKF, 128), lambda i: (i, 0)),
              pl.BlockSpec((BLKF, 128), lambda i: (i, 0)),
              pl.BlockSpec((8, 128), lambda i: (0, 0)),
              pl.BlockSpec((128, 128), lambda i: (0, 0)),
              pl.BlockSpec((128, 128), lambda i: (0, 0)),
              pl.BlockSpec((128, 128), lambda i: (0, 0)),
              pl.BlockSpec((128, 128), lambda i: (0, 0))],
    out_specs=[pl.BlockSpec((BLKF, 128), lambda i: (i, 0)),
               pl.BlockSpec((BLKF, 128), lambda i: (i, 0)),
               pl.BlockSpec((BLKF, 8), lambda i: (i, 0))],
    out_shape=[jax.ShapeDtypeStruct((FLAT, 128), jnp.float32),
               jax.ShapeDtypeStruct((FLAT, 128), jnp.float32),
               jax.ShapeDtypeStruct((FLAT, 8), jnp.float32)],
)


def _mm2_body(s0_ref, s1_ref, glo_ref, ghi_ref, dinv_ref, e8_ref, blo_ref,
              bhi_ref, w2a_ref, w2b_ref, w2c_ref, w2d_ref, olo_ref, ohi_ref):
    dinvf = jnp.dot(dinv_ref[...], e8_ref[...],
                    preferred_element_type=jnp.float32)
    hl = jnp.maximum(dinvf * (s0_ref[...] + glo_ref[...]) + blo_ref[...], 0.0)
    hh = jnp.maximum(dinvf * (s1_ref[...] + ghi_ref[...]) + bhi_ref[...], 0.0)
    olo_ref[...] = (jnp.dot(hl, w2a_ref[...], preferred_element_type=jnp.float32)
                    + jnp.dot(hh, w2b_ref[...],
                              preferred_element_type=jnp.float32)) * dinvf
    ohi_ref[...] = (jnp.dot(hl, w2c_ref[...], preferred_element_type=jnp.float32)
                    + jnp.dot(hh, w2d_ref[...],
                              preferred_element_type=jnp.float32)) * dinvf


_mm2 = pl.pallas_call(
    _mm2_body,
    grid=(FLAT // BLKF,),
    in_specs=[pl.BlockSpec((BLKF, 128), lambda i: (i, 0)),
              pl.BlockSpec((BLKF, 128), lambda i: (i, 0)),
              pl.BlockSpec((BLKF, 128), lambda i: (i, 0)),
              pl.BlockSpec((BLKF, 128), lambda i: (i, 0)),
              pl.BlockSpec((BLKF, 8), lambda i: (i, 0)),
              pl.BlockSpec((8, 128), lambda i: (0, 0)),
              pl.BlockSpec((1, 128), lambda i: (0, 0)),
              pl.BlockSpec((1, 128), lambda i: (0, 0)),
              pl.BlockSpec((128, 128), lambda i: (0, 0)),
              pl.BlockSpec((128, 128), lambda i: (0, 0)),
              pl.BlockSpec((128, 128), lambda i: (0, 0)),
              pl.BlockSpec((128, 128), lambda i: (0, 0))],
    out_specs=[pl.BlockSpec((BLKF, 128), lambda i: (i, 0)),
               pl.BlockSpec((BLKF, 128), lambda i: (i, 0))],
    out_shape=[jax.ShapeDtypeStruct((FLAT, 128), jnp.float32),
               jax.ShapeDtypeStruct((FLAT, 128), jnp.float32)],
)


def _mm3_body(s0_ref, s1_ref, glo_ref, ghi_ref, dinv_ref, b2_ref, batch_ref,
              wc_ref, bc_ref, out_ref, sums, cnts):
    i = pl.program_id(0)

    @pl.when(i == 0)
    def _():
        sums[...] = jnp.zeros_like(sums)
        cnts[...] = jnp.zeros_like(cnts)

    dinv = dinv_ref[...]
    b = b2_ref[...]
    hl = jnp.maximum(dinv * (s0_ref[...] + glo_ref[...]) + b[:, :HALF], 0.0)
    hh = jnp.maximum(dinv * (s1_ref[...] + ghi_ref[...]) + b[:, HALF:], 0.0)
    h = jnp.concatenate([hl, hh], axis=1)
    bb = batch_ref[...]
    gid = lax.broadcasted_iota(jnp.int32, (BLK3, N_GRAPHS), 1)
    a = (bb == gid).astype(jnp.float32)
    sums[...] += lax.dot_general(a, h, (((0,), (0,)), ((), ())),
                                 preferred_element_type=jnp.float32)
    cnts[...] += jnp.sum(a, axis=0)[:, None]

    @pl.when(i == N_NODES // BLK3 - 1)
    def _():
        pooled = sums[...] / jnp.maximum(cnts[...], 1.0)
        out_ref[...] = jnp.dot(pooled, wc_ref[...],
                               preferred_element_type=jnp.float32) + bc_ref[...]


_mm3 = pl.pallas_call(
    _mm3_body,
    grid=(N_NODES // BLK3,),
    in_specs=[pl.BlockSpec((BLK3, HALF), lambda i: (i, 0)),
              pl.BlockSpec((BLK3, HALF), lambda i: (i, 0)),
              pl.BlockSpec((BLK3, HALF), lambda i: (i, 0)),
              pl.BlockSpec((BLK3, HALF), lambda i: (i, 0)),
              pl.BlockSpec((BLK3, 1), lambda i: (i, 0)),
              pl.BlockSpec((1, HIDDEN), lambda i: (0, 0)),
              pl.BlockSpec((BLK3, 1), lambda i: (i, 0)),
              pl.BlockSpec((HIDDEN, CLASSES), lambda i: (0, 0)),
              pl.BlockSpec((1, CLASSES), lambda i: (0, 0))],
    out_specs=pl.BlockSpec((N_GRAPHS, CLASSES), lambda i: (0, 0)),
    out_shape=jax.ShapeDtypeStruct((N_GRAPHS, CLASSES), jnp.float32),
    scratch_shapes=[pltpu.VMEM((N_GRAPHS, HIDDEN), jnp.float32),
                    pltpu.VMEM((N_GRAPHS, 1), jnp.float32)],
    compiler_params=pltpu.CompilerParams(
        dimension_semantics=("arbitrary",)),
)


def kernel(x, edge_index, batch, emb, W1, b1, W2, b2, Wc, bc):
    f32 = jnp.float32
    x2 = jnp.pad(x, (0, N_PAD - N_NODES)).reshape(N_PAD // RW, RW)
    ei = jnp.pad(edge_index, ((0, 0), (0, E_PAD - N_EDGES)),
                 constant_values=N_NODES)
    src2 = ei[0].reshape(EROWS, RW)
    dst2 = ei[1].reshape(EROWS, RW)
    embv = emb.reshape(2 * VOCAB_ROWS, HALF)

    i8 = jnp.eye(8, dtype=f32)
    e8 = jnp.kron(i8, jnp.ones((1, HALF), f32))
    w1k = [jnp.kron(i8, W1[:HALF, :HALF]), jnp.kron(i8, W1[HALF:, :HALF]),
           jnp.kron(i8, W1[:HALF, HALF:]), jnp.kron(i8, W1[HALF:, HALF:])]
    w2k = [jnp.kron(i8, W2[:HALF, :HALF]), jnp.kron(i8, W2[HALF:, :HALF]),
           jnp.kron(i8, W2[:HALF, HALF:]), jnp.kron(i8, W2[HALF:, HALF:])]
    b1lo = jnp.tile(b1[:HALF], 8)[None, :]
    b1hi = jnp.tile(b1[HALF:], 8)[None, :]

    h0lo, h0hi, deg_a, deg_b, _unused_degp = _kpre(x2, ei[1], embv)
    glo_f, ghi_f, dinv8 = _mm1(deg_a.reshape(FLAT, 8), deg_b.reshape(FLAT, 8),
                               h0lo.reshape(FLAT, 128), h0hi.reshape(FLAT, 128),
                               e8, *w1k)
    glo = glo_f.reshape(N_PAD, HALF)
    ghi = ghi_f.reshape(N_PAD, HALF)
    s0, s1 = _kedge(src2, dst2, glo, ghi)
    g2lo_f, g2hi_f = _mm2(s0.reshape(FLAT, 128), s1.reshape(FLAT, 128),
                          glo_f, ghi_f, dinv8, e8, b1lo, b1hi, *w2k)
    g2lo = g2lo_f.reshape(N_PAD, HALF)
    g2hi = g2hi_f.reshape(N_PAD, HALF)
    t0, t1 = _kedge(src2, dst2, g2lo, g2hi)
    out = _mm3(t0, t1, g2lo, g2hi, dinv8.reshape(N_PAD, 1),
               b2.reshape(1, HIDDEN), batch.reshape(N_NODES, 1), Wc,
               bc.reshape(1, CLASSES))
    return out


# speedup vs baseline: 1.5272x; 1.1313x over previous
import functools

import jax
import jax.numpy as jnp
from jax import lax
from jax.experimental import pallas as pl
from jax.experimental.pallas import tpu as pltpu
from jax.experimental.pallas import tpu_sc as plsc

N_NODES = 100000
N_EDGES = 1600000
EMBED = 32
HIDDEN = 32
CLASSES = 16
N_GRAPHS = 512
HALF = 16

N_PAD = 102400
E_PAD = 1605632
VOCAB_ROWS = 100000

NC, NS = 2, 16
NW = NC * NS
LANES = 16

RW = 128
TOK_PER_TILE = N_PAD // NW
TOK_ROWS = TOK_PER_TILE // RW
EDG_PER_TILE_DEG = E_PAD // NW
DEG_CHUNK = 3136
EROWS = E_PAD // RW
EROWS_PER_TILE = EROWS // NS
SUP = 4
NSUP = EROWS_PER_TILE // SUP
NODES_PER_TILE = N_PAD // NS
WB_CHUNK = 320

_sc_mesh = plsc.VectorSubcoreMesh(core_axis_name="c", subcore_axis_name="s")
_sc_params = pltpu.CompilerParams(use_tc_tiling_on_sc=False,
                                  needs_layout_passes=False)


@functools.partial(
    pl.kernel,
    out_type=[jax.ShapeDtypeStruct((N_PAD, HALF), jnp.float32),
              jax.ShapeDtypeStruct((N_PAD, HALF), jnp.float32),
              jax.ShapeDtypeStruct((N_PAD,), jnp.float32),
              jax.ShapeDtypeStruct((N_PAD,), jnp.float32),
              jax.ShapeDtypeStruct((NW, N_PAD), jnp.float32)],
    mesh=_sc_mesh,
    scratch_types=[
        pltpu.VMEM((TOK_ROWS, RW), jnp.int32),
        pltpu.VMEM((TOK_ROWS, RW), jnp.int32),
        pltpu.VMEM((TOK_ROWS, RW), jnp.int32),
        pltpu.VMEM((2 * RW, HALF), jnp.float32),
        pltpu.VMEM((2 * RW, HALF), jnp.float32),
        pltpu.VMEM((DEG_CHUNK, ), jnp.int32),
        pltpu.VMEM((N_PAD,), jnp.float32),
        pltpu.VMEM((NODES_PER_TILE,), jnp.float32),
        pltpu.SemaphoreType.DMA,
        pltpu.SemaphoreType.DMA,
    ],
    compiler_params=_sc_params,
)
def _kpre(x2, dst_flat, embv, h0lo, h0hi, deg_a, deg_b, degp, xb, xlo, xhi,
          rowsl, rowsh, dstb, degv, rowb, gsem, wsem):
    c = lax.axis_index("c")
    s = lax.axis_index("s")
    w = c * NS + s

    pltpu.sync_copy(x2.at[pl.ds(w * TOK_ROWS, TOK_ROWS)], xb)

    def idx_body(k, carry):
        def lane_body(j, c2):
            sl = pl.ds(j * LANES, LANES)
            v2 = xb[k, sl] * 2
            xlo[k, sl] = v2
            xhi[k, sl] = v2 + 1
            return c2

        lax.fori_loop(0, RW // LANES, lane_body, 0)
        return carry

    lax.fori_loop(0, TOK_ROWS, idx_body, 0)

    g0 = pltpu.async_copy(embv.at[xlo.at[0]], rowsl.at[pl.ds(0, RW)], gsem)
    g1 = pltpu.async_copy(embv.at[xhi.at[0]], rowsh.at[pl.ds(0, RW)], gsem)

    def zero_body(i, carry):
        degv[pl.ds(i * LANES, LANES)] = jnp.zeros((LANES,), jnp.float32)
        return carry

    lax.fori_loop(0, N_PAD // LANES, zero_body, 0)
    g0.wait()
    g1.wait()

    def body(k, carry):
        p = lax.rem(k, 2)
        hsl = pl.ds(w * TOK_PER_TILE + k * RW, RW)
        bsl = pl.ds(p * RW, RW)
        nsl = pl.ds((1 - p) * RW, RW)

        @pl.when(k < TOK_ROWS - 1)
        def _():
            pltpu.async_copy(embv.at[xlo.at[k + 1]], rowsl.at[nsl], gsem)
            pltpu.async_copy(embv.at[xhi.at[k + 1]], rowsh.at[nsl], gsem)

        pltpu.async_copy(rowsl.at[bsl], h0lo.at[hsl], wsem)
        pltpu.async_copy(rowsh.at[bsl], h0hi.at[hsl], wsem)

        @pl.when(k < TOK_ROWS - 1)
        def _():
            pltpu.make_async_copy(embv.at[xlo.at[k + 1]], rowsl.at[nsl],
                                  gsem).wait()
            pltpu.make_async_copy(embv.at[xhi.at[k + 1]], rowsh.at[nsl],
                                  gsem).wait()

        pltpu.make_async_copy(rowsl.at[bsl], h0lo.at[hsl], wsem).wait()
        pltpu.make_async_copy(rowsh.at[bsl], h0hi.at[hsl], wsem).wait()
        return carry

    lax.fori_loop(0, TOK_ROWS, body, 0)

    ones = jnp.ones((LANES,), jnp.float32)

    def chunk_body(t, carry):
        pltpu.sync_copy(
            dst_flat.at[pl.ds(w * EDG_PER_TILE_DEG + t * DEG_CHUNK, DEG_CHUNK)],
            dstb)

        def vec_body(i, c2):
            plsc.addupdate_scatter(degv, [dstb[pl.ds(i * LANES, LANES)]], ones)
            return c2

        lax.fori_loop(0, DEG_CHUNK // LANES, vec_body, 0)
        return carry

    lax.fori_loop(0, EDG_PER_TILE_DEG // DEG_CHUNK, chunk_body, 0)

    pltpu.sync_copy(degv, degp.at[w])
    plsc.subcore_barrier()
    base = s * NODES_PER_TILE
    pltpu.sync_copy(degp.at[c * NS, pl.ds(base, NODES_PER_TILE)],
                    degv.at[pl.ds(0, NODES_PER_TILE)])
    for t in range(1, NS):
        pltpu.sync_copy(degp.at[c * NS + t, pl.ds(base, NODES_PER_TILE)], rowb)

        def add_body(i, carry):
            sl = pl.ds(i * LANES, LANES)
            degv[sl] = degv[sl] + rowb[sl]
            return carry

        lax.fori_loop(0, NODES_PER_TILE // LANES, add_body, 0)

    @pl.when(c == 0)
    def _():
        pltpu.sync_copy(degv.at[pl.ds(0, NODES_PER_TILE)],
                        deg_a.at[pl.ds(base, NODES_PER_TILE)])

    @pl.when(c == 1)
    def _():
        pltpu.sync_copy(degv.at[pl.ds(0, NODES_PER_TILE)],
                        deg_b.at[pl.ds(base, NODES_PER_TILE)])


def _edge_phase(g_ref, out_ref, acc, srcb, dstb, rows, zbuf, gsem, ssem,
                src2, dst2, s):
    def zb(i, carry):
        zbuf[i, :] = jnp.zeros((LANES,), jnp.float32)
        return carry

    lax.fori_loop(0, WB_CHUNK, zb, 0)
    for k in range(NODES_PER_TILE // WB_CHUNK):
        pltpu.sync_copy(
            zbuf, acc.at[pl.ds(s * NODES_PER_TILE + k * WB_CHUNK, WB_CHUNK)])
    plsc.subcore_barrier()

    base_r = s * EROWS_PER_TILE
    pltpu.sync_copy(src2.at[pl.ds(base_r, SUP)], srcb.at[pl.ds(0, SUP)])
    pltpu.sync_copy(dst2.at[pl.ds(base_r, SUP)], dstb.at[pl.ds(0, SUP)])

    def sup_body(t, carry):
        p = lax.rem(t, 2)
        q = 1 - p
        gds = [pltpu.async_copy(g_ref.at[srcb.at[p * SUP + j]],
                                rows.at[pl.ds((p * SUP + j) * RW, RW)], gsem)
               for j in range(SUP)]

        @pl.when(t >= 1)
        def _():
            for j in range(SUP):
                pltpu.make_async_copy(
                    rows.at[pl.ds((q * SUP + j) * RW, RW)],
                    acc.at[dstb.at[q * SUP + j]], ssem).wait()

        @pl.when(t < NSUP - 1)
        def _():
            r1 = base_r + (t + 1) * SUP
            pltpu.sync_copy(src2.at[pl.ds(r1, SUP)],
                            srcb.at[pl.ds(q * SUP, SUP)])
            pltpu.sync_copy(dst2.at[pl.ds(r1, SUP)],
                            dstb.at[pl.ds(q * SUP, SUP)])

        for d in gds:
            d.wait()
        for j in range(SUP):
            pltpu.async_copy(rows.at[pl.ds((p * SUP + j) * RW, RW)],
                             acc.at[dstb.at[p * SUP + j]], ssem, add=True)
        return carry

    lax.fori_loop(0, NSUP, sup_body, 0)
    qf = (NSUP - 1) % 2
    for j in range(SUP):
        pltpu.make_async_copy(rows.at[pl.ds((qf * SUP + j) * RW, RW)],
                              acc.at[dstb.at[qf * SUP + j]], ssem).wait()
    plsc.subcore_barrier()

    for k in range(NODES_PER_TILE // WB_CHUNK):
        sl = pl.ds(s * NODES_PER_TILE + k * WB_CHUNK, WB_CHUNK)
        pltpu.sync_copy(acc.at[sl], zbuf)
        pltpu.sync_copy(zbuf, out_ref.at[sl])


@functools.partial(
    pl.kernel,
    out_type=[jax.ShapeDtypeStruct((N_PAD, HALF), jnp.float32),
              jax.ShapeDtypeStruct((N_PAD, HALF), jnp.float32)],
    mesh=_sc_mesh,
    scratch_types=[
        pltpu.VMEM_SHARED((N_PAD, HALF), jnp.float32),
        pltpu.VMEM((2 * SUP, RW), jnp.int32),
        pltpu.VMEM((2 * SUP, RW), jnp.int32),
        pltpu.VMEM((2 * SUP * RW, HALF), jnp.float32),
        pltpu.VMEM((WB_CHUNK, HALF), jnp.float32),
        pltpu.SemaphoreType.DMA,
        pltpu.SemaphoreType.DMA,
    ],
    compiler_params=_sc_params,
)
def _kedge(src2, dst2, g0, g1, s0, s1, acc, srcb, dstb, rows, zbuf, gsem, ssem):
    c = lax.axis_index("c")
    s = lax.axis_index("s")

    @pl.when(c == 0)
    def _():
        _edge_phase(g0, s0, acc, srcb, dstb, rows, zbuf, gsem, ssem,
                    src2, dst2, s)

    @pl.when(c == 1)
    def _():
        _edge_phase(g1, s1, acc, srcb, dstb, rows, zbuf, gsem, ssem,
                    src2, dst2, s)


FLAT = N_PAD // 8
BLKF = FLAT // 4
BLK3 = 4000


def _mm1_body(dega_ref, degb_ref, h0lo_ref, h0hi_ref, e8_ref, w1a_ref,
              w1b_ref, w1c_ref, w1d_ref, glo_ref, ghi_ref, dinv_ref):
    deg = dega_ref[...] + degb_ref[...] + 1.0
    dinv8 = lax.rsqrt(deg)
    dinvf = jnp.dot(dinv8, e8_ref[...], preferred_element_type=jnp.float32)
    hl = h0lo_ref[...]
    hh = h0hi_ref[...]
    glo_ref[...] = (jnp.dot(hl, w1a_ref[...], preferred_element_type=jnp.float32)
                    + jnp.dot(hh, w1b_ref[...],
                              preferred_element_type=jnp.float32)) * dinvf
    ghi_ref[...] = (jnp.dot(hl, w1c_ref[...], preferred_element_type=jnp.float32)
                    + jnp.dot(hh, w1d_ref[...],
                              preferred_element_type=jnp.float32)) * dinvf
    dinv_ref[...] = dinv8


_mm1 = pl.pallas_call(
    _mm1_body,
    grid=(FLAT // BLKF,),
    in_specs=[pl.BlockSpec((BLKF, 8), lambda i: (i, 0)),
              pl.BlockSpec((BLKF, 8), lambda i: (i, 0)),
              pl.BlockSpec((BLKF, 128), lambda i: (i, 0)),
              pl.BlockSpec((BLKF, 128), lambda i: (i, 0)),
              pl.BlockSpec((8, 128), lambda i: (0, 0)),
              pl.BlockSpec((128, 128), lambda i: (0, 0)),
              pl.BlockSpec((128, 128), lambda i: (0, 0)),
              pl.BlockSpec((128, 128), lambda i: (0, 0)),
              pl.BlockSpec((128, 128), lambda i: (0, 0))],
    out_specs=[pl.BlockSpec((BLKF, 128), lambda i: (i, 0)),
               pl.BlockSpec((BLKF, 128), lambda i: (i, 0)),
               pl.BlockSpec((BLKF, 8), lambda i: (i, 0))],
    out_shape=[jax.ShapeDtypeStruct((FLAT, 128), jnp.float32),
               jax.ShapeDtypeStruct((FLAT, 128), jnp.float32),
               jax.ShapeDtypeStruct((FLAT, 8), jnp.float32)],
)


def _mm2_body(s0_ref, s1_ref, glo_ref, ghi_ref, dinv_ref, e8_ref, blo_ref,
              bhi_ref, w2a_ref, w2b_ref, w2c_ref, w2d_ref, olo_ref, ohi_ref):
    dinvf = jnp.dot(dinv_ref[...], e8_ref[...],
                    preferred_element_type=jnp.float32)
    hl = jnp.maximum(dinvf * (s0_ref[...] + glo_ref[...]) + blo_ref[...], 0.0)
    hh = jnp.maximum(dinvf * (s1_ref[...] + ghi_ref[...]) + bhi_ref[...], 0.0)
    olo_ref[...] = (jnp.dot(hl, w2a_ref[...], preferred_element_type=jnp.float32)
                    + jnp.dot(hh, w2b_ref[...],
                              preferred_element_type=jnp.float32)) * dinvf
    ohi_ref[...] = (jnp.dot(hl, w2c_ref[...], preferred_element_type=jnp.float32)
                    + jnp.dot(hh, w2d_ref[...],
                              preferred_element_type=jnp.float32)) * dinvf


_mm2 = pl.pallas_call(
    _mm2_body,
    grid=(FLAT // BLKF,),
    in_specs=[pl.BlockSpec((BLKF, 128), lambda i: (i, 0)),
              pl.BlockSpec((BLKF, 128), lambda i: (i, 0)),
              pl.BlockSpec((BLKF, 128), lambda i: (i, 0)),
              pl.BlockSpec((BLKF, 128), lambda i: (i, 0)),
              pl.BlockSpec((BLKF, 8), lambda i: (i, 0)),
              pl.BlockSpec((8, 128), lambda i: (0, 0)),
              pl.BlockSpec((1, 128), lambda i: (0, 0)),
              pl.BlockSpec((1, 128), lambda i: (0, 0)),
              pl.BlockSpec((128, 128), lambda i: (0, 0)),
              pl.BlockSpec((128, 128), lambda i: (0, 0)),
              pl.BlockSpec((128, 128), lambda i: (0, 0)),
              pl.BlockSpec((128, 128), lambda i: (0, 0))],
    out_specs=[pl.BlockSpec((BLKF, 128), lambda i: (i, 0)),
               pl.BlockSpec((BLKF, 128), lambda i: (i, 0))],
    out_shape=[jax.ShapeDtypeStruct((FLAT, 128), jnp.float32),
               jax.ShapeDtypeStruct((FLAT, 128), jnp.float32)],
)


BLKF3 = 3200


def _mm3_body(s0_ref, s1_ref, glo_ref, ghi_ref, dinv_ref, e8_ref, blo_ref,
              bhi_ref, batch_ref, wc_ref, bc_ref, out_ref, slo, shi, cnts):
    i = pl.program_id(0)

    @pl.when(i == 0)
    def _():
        slo[...] = jnp.zeros_like(slo)
        shi[...] = jnp.zeros_like(shi)
        cnts[...] = jnp.zeros_like(cnts)

    dinvf = jnp.dot(dinv_ref[...], e8_ref[...],
                    preferred_element_type=jnp.float32)
    hl = jnp.maximum(dinvf * (s0_ref[...] + glo_ref[...]) + blo_ref[...], 0.0)
    hh = jnp.maximum(dinvf * (s1_ref[...] + ghi_ref[...]) + bhi_ref[...], 0.0)
    bb = batch_ref[...]
    gid = lax.broadcasted_iota(jnp.int32, (BLKF3, N_GRAPHS), 1)
    dn = (((0,), (0,)), ((), ()))
    for q in range(8):
        aq = (bb[:, q:q + 1] == gid).astype(jnp.float32)
        slo[...] += lax.dot_general(aq, hl[:, q * HALF:(q + 1) * HALF], dn,
                                    preferred_element_type=jnp.float32)
        shi[...] += lax.dot_general(aq, hh[:, q * HALF:(q + 1) * HALF], dn,
                                    preferred_element_type=jnp.float32)
        cnts[...] += jnp.sum(aq, axis=0)[:, None]

    @pl.when(i == FLAT // BLKF3 - 1)
    def _():
        inv = 1.0 / jnp.maximum(cnts[...], 1.0)
        pooled = jnp.concatenate([slo[...], shi[...]], axis=1) * inv
        out_ref[...] = jnp.dot(pooled, wc_ref[...],
                               preferred_element_type=jnp.float32) + bc_ref[...]


_mm3 = pl.pallas_call(
    _mm3_body,
    grid=(FLAT // BLKF3,),
    in_specs=[pl.BlockSpec((BLKF3, 128), lambda i: (i, 0)),
              pl.BlockSpec((BLKF3, 128), lambda i: (i, 0)),
              pl.BlockSpec((BLKF3, 128), lambda i: (i, 0)),
              pl.BlockSpec((BLKF3, 128), lambda i: (i, 0)),
              pl.BlockSpec((BLKF3, 8), lambda i: (i, 0)),
              pl.BlockSpec((8, 128), lambda i: (0, 0)),
              pl.BlockSpec((1, 128), lambda i: (0, 0)),
              pl.BlockSpec((1, 128), lambda i: (0, 0)),
              pl.BlockSpec((BLKF3, 8), lambda i: (i, 0)),
              pl.BlockSpec((HIDDEN, CLASSES), lambda i: (0, 0)),
              pl.BlockSpec((1, CLASSES), lambda i: (0, 0))],
    out_specs=pl.BlockSpec((N_GRAPHS, CLASSES), lambda i: (0, 0)),
    out_shape=jax.ShapeDtypeStruct((N_GRAPHS, CLASSES), jnp.float32),
    scratch_shapes=[pltpu.VMEM((N_GRAPHS, HALF), jnp.float32),
                    pltpu.VMEM((N_GRAPHS, HALF), jnp.float32),
                    pltpu.VMEM((N_GRAPHS, 1), jnp.float32)],
    compiler_params=pltpu.CompilerParams(
        dimension_semantics=("arbitrary",)),
)


def kernel(x, edge_index, batch, emb, W1, b1, W2, b2, Wc, bc):
    f32 = jnp.float32
    x2 = jnp.pad(x, (0, N_PAD - N_NODES)).reshape(N_PAD // RW, RW)
    ei = jnp.pad(edge_index, ((0, 0), (0, E_PAD - N_EDGES)),
                 constant_values=N_NODES)
    src2 = ei[0].reshape(EROWS, RW)
    dst2 = ei[1].reshape(EROWS, RW)
    embv = emb.reshape(2 * VOCAB_ROWS, HALF)

    i8 = jnp.eye(8, dtype=f32)
    e8 = jnp.kron(i8, jnp.ones((1, HALF), f32))
    w1k = [jnp.kron(i8, W1[:HALF, :HALF]), jnp.kron(i8, W1[HALF:, :HALF]),
           jnp.kron(i8, W1[:HALF, HALF:]), jnp.kron(i8, W1[HALF:, HALF:])]
    w2k = [jnp.kron(i8, W2[:HALF, :HALF]), jnp.kron(i8, W2[HALF:, :HALF]),
           jnp.kron(i8, W2[:HALF, HALF:]), jnp.kron(i8, W2[HALF:, HALF:])]
    b1lo = jnp.tile(b1[:HALF], 8)[None, :]
    b1hi = jnp.tile(b1[HALF:], 8)[None, :]
    b2lo = jnp.tile(b2[:HALF], 8)[None, :]
    b2hi = jnp.tile(b2[HALF:], 8)[None, :]

    h0lo, h0hi, deg_a, deg_b, _unused_degp = _kpre(x2, ei[1], embv)
    glo_f, ghi_f, dinv8 = _mm1(deg_a.reshape(FLAT, 8), deg_b.reshape(FLAT, 8),
                               h0lo.reshape(FLAT, 128), h0hi.reshape(FLAT, 128),
                               e8, *w1k)
    glo = glo_f.reshape(N_PAD, HALF)
    ghi = ghi_f.reshape(N_PAD, HALF)
    s0, s1 = _kedge(src2, dst2, glo, ghi)
    g2lo_f, g2hi_f = _mm2(s0.reshape(FLAT, 128), s1.reshape(FLAT, 128),
                          glo_f, ghi_f, dinv8, e8, b1lo, b1hi, *w2k)
    g2lo = g2lo_f.reshape(N_PAD, HALF)
    g2hi = g2hi_f.reshape(N_PAD, HALF)
    t0, t1 = _kedge(src2, dst2, g2lo, g2hi)
    out = _mm3(t0.reshape(FLAT, 128), t1.reshape(FLAT, 128), g2lo_f, g2hi_f,
               dinv8, e8, b2lo, b2hi,
               jnp.pad(batch, (0, N_PAD - N_NODES),
                       constant_values=N_GRAPHS).reshape(FLAT, 8), Wc,
               bc.reshape(1, CLASSES))
    return out
